# baseline (speedup 1.0000x reference)
import functools

import jax
import jax.numpy as jnp
from jax import lax
from jax.experimental import pallas as pl
from jax.experimental.pallas import tpu as pltpu
from jax.experimental.pallas import tpu_sc as plsc

V = 1000000
B = 4096
K = 20
D = 16
S = K + 1

NC = 2
NS = 16
NW = NC * NS
BW = B // NW
NG = BW // 16
NR = BW * D // 128


def _make_main_sc():
    mesh = plsc.VectorSubcoreMesh(core_axis_name="c", subcore_axis_name="s")

    @functools.partial(
        pl.kernel,
        mesh=mesh,
        out_type=[
            jax.ShapeDtypeStruct((NW, BW), jnp.float32),
            jax.ShapeDtypeStruct((NW, BW), jnp.float32),
        ],
        scratch_types=[
            pltpu.VMEM((BW,), jnp.int32),
            pltpu.VMEM((S, BW), jnp.int32),
            pltpu.VMEM((NR, 128), jnp.int32),
            pltpu.VMEM((NR, 128), jnp.int32),
            pltpu.VMEM((NR, 128), jnp.int32),
            pltpu.VMEM((NR, 128), jnp.float32),
            pltpu.VMEM((NR, 128), jnp.float32),
            pltpu.VMEM((NR, 128), jnp.float32),
            pltpu.VMEM((BW,), jnp.float32),
            pltpu.VMEM((BW,), jnp.float32),
            pltpu.SemaphoreType.DMA,
            pltpu.SemaphoreType.DMA,
            pltpu.SemaphoreType.DMA,
        ],
        compiler_params=pltpu.CompilerParams(needs_layout_passes=False),
    )
    def _main(ih_flat, hu_flat, ii_hbm, hui_hbm, den_out, sc_out,
              ii_v, hui_v, idxe, idx0, idx1, gie, gb0, gb1,
              den_v, sc_v, sem_ie, sem0, sem1):
        w = lax.axis_index("s") * NC + lax.axis_index("c")
        pltpu.sync_copy(ii_hbm.at[w], ii_v)
        pltpu.sync_copy(hui_hbm.at[w], hui_v)

        def build_idx(idx_ref, vec_of):
            def bc(c, _):
                base = vec_of(c)
                for d in range(D):
                    row = 2 * c + d // 8
                    col = (d % 8) * 16
                    idx_ref[row, pl.ds(col, 16)] = base + d * V
                return 0
            lax.fori_loop(0, NG, bc, 0)

        def fire(table, idx_ref, buf, sem):
            return [
                pltpu.async_copy(table.at[idx_ref.at[g]], buf.at[g], sem)
                for g in range(NR)
            ]

        build_idx(idxe, lambda c: ii_v[pl.ds(c * 16, 16)])
        cp_ie = fire(ih_flat, idxe, gie, sem_ie)
        idxs = (idx0, idx1)
        bufs = (gb0, gb1)
        sems = (sem0, sem1)
        build_idx(idxs[0], lambda c: hui_v[0, pl.ds(c * 16, 16)])
        cps = {0: fire(hu_flat, idxs[0], bufs[0], sems[0])}
        for c in cp_ie:
            c.wait()

        for j in range(S):
            if j + 1 < S:
                nb = (j + 1) % 2
                build_idx(idxs[nb],
                          lambda c, _j=j: hui_v[_j + 1, pl.ds(c * 16, 16)])
                cps[j + 1] = fire(hu_flat, idxs[nb], bufs[nb], sems[nb])
            for c in cps[j]:
                c.wait()
            buf = bufs[j % 2]

            def score_grp(c, _):
                sl = pl.ds(c * 16, 16)
                s = buf[2 * c, pl.ds(0, 16)] * gie[2 * c, pl.ds(0, 16)]
                for d in range(1, D):
                    row = 2 * c + d // 8
                    col = (d % 8) * 16
                    s = s + buf[row, pl.ds(col, 16)] * gie[row, pl.ds(col, 16)]
                if j == 0:
                    sc_v[sl] = s
                elif j == 1:
                    den_v[sl] = jnp.exp(s)
                else:
                    den_v[sl] = den_v[sl] + jnp.exp(s)
                return 0
            lax.fori_loop(0, NG, score_grp, 0)

        pltpu.sync_copy(den_v, den_out.at[w])
        pltpu.sync_copy(sc_v, sc_out.at[w])

    return _main


def _finish_body(den_ref, sc_ref, out_ref):
    nll = (jnp.sum(jnp.log(den_ref[...])) - jnp.sum(sc_ref[...])) / B
    out_ref[0, 0] = nll


def kernel(inputs, predict, normal, I_H, H_U):
    ih_flat = I_H.T.reshape(V * D)
    hu_flat = H_U.T.reshape(V * D)
    ii = inputs.reshape(NW, BW).astype(jnp.int32)
    hui = jnp.concatenate(
        [predict.reshape(B, 1), normal.reshape(B, K)], axis=1
    ).astype(jnp.int32)
    hui = hui.reshape(NW, BW, S).transpose(0, 2, 1)
    den, scp = _make_main_sc()(ih_flat, hu_flat, ii, hui)
    out = pl.pallas_call(
        _finish_body,
        out_shape=jax.ShapeDtypeStruct((1, 1), jnp.float32),
        out_specs=pl.BlockSpec(memory_space=pltpu.SMEM),
    )(den, scp)
    return out.reshape(1)


# speedup vs baseline: 1.0001x; 1.0001x over previous
import functools

import jax
import jax.numpy as jnp
from jax import lax
from jax.experimental import pallas as pl
from jax.experimental.pallas import tpu as pltpu
from jax.experimental.pallas import tpu_sc as plsc

V = 1000000
B = 4096
K = 20
D = 16
S = K + 1

NC = 2
NS = 16
NW = NC * NS
BW = B // NW
NG = BW // 16
NR = BW * D // 128


def _make_main_sc():
    mesh = plsc.VectorSubcoreMesh(core_axis_name="c", subcore_axis_name="s")

    @functools.partial(
        pl.kernel,
        mesh=mesh,
        out_type=[
            jax.ShapeDtypeStruct((NW, BW), jnp.float32),
            jax.ShapeDtypeStruct((NW, BW), jnp.float32),
        ],
        scratch_types=[
            pltpu.VMEM((BW,), jnp.int32),
            pltpu.VMEM((S, BW), jnp.int32),
            pltpu.VMEM((BW * D,), jnp.int32),
            pltpu.VMEM((BW * D,), jnp.int32),
            pltpu.VMEM((BW * D,), jnp.int32),
            pltpu.VMEM((BW * D,), jnp.float32),
            pltpu.VMEM((BW * D,), jnp.float32),
            pltpu.VMEM((BW * D,), jnp.float32),
            pltpu.VMEM((BW,), jnp.float32),
            pltpu.VMEM((BW,), jnp.float32),
            pltpu.SemaphoreType.DMA,
            pltpu.SemaphoreType.DMA,
            pltpu.SemaphoreType.DMA,
        ],
        compiler_params=pltpu.CompilerParams(needs_layout_passes=False),
    )
    def _main(ih_flat, hu_flat, ii_hbm, hui_hbm, den_out, sc_out,
              ii_v, hui_v, idxe, idx0, idx1, gie, gb0, gb1,
              den_v, sc_v, sem_ie, sem0, sem1):
        w = lax.axis_index("s") * NC + lax.axis_index("c")
        pltpu.sync_copy(ii_hbm.at[w], ii_v)
        pltpu.sync_copy(hui_hbm.at[w], hui_v)

        def build_idx(idx_ref, vec_of):
            def bc(c, _):
                base = vec_of(c)
                for d in range(D):
                    idx_ref[pl.ds(c * 256 + d * 16, 16)] = base + d * V
                return 0
            lax.fori_loop(0, NG, bc, 0)

        def fire(table, idx_ref, buf, sem):
            return [pltpu.async_copy(table.at[idx_ref], buf, sem)]

        build_idx(idxe, lambda c: ii_v[pl.ds(c * 16, 16)])
        cp_ie = fire(ih_flat, idxe, gie, sem_ie)
        idxs = (idx0, idx1)
        bufs = (gb0, gb1)
        sems = (sem0, sem1)
        build_idx(idxs[0], lambda c: hui_v[0, pl.ds(c * 16, 16)])
        cps = {0: fire(hu_flat, idxs[0], bufs[0], sems[0])}
        for c in cp_ie:
            c.wait()

        for j in range(S):
            if j + 1 < S:
                nb = (j + 1) % 2
                build_idx(idxs[nb],
                          lambda c, _j=j: hui_v[_j + 1, pl.ds(c * 16, 16)])
                cps[j + 1] = fire(hu_flat, idxs[nb], bufs[nb], sems[nb])
            for c in cps[j]:
                c.wait()
            buf = bufs[j % 2]

            def score_grp(c, _):
                sl = pl.ds(c * 16, 16)
                p0 = pl.ds(c * 256, 16)
                s = buf[p0] * gie[p0]
                for d in range(1, D):
                    p = pl.ds(c * 256 + d * 16, 16)
                    s = s + buf[p] * gie[p]
                if j == 0:
                    sc_v[sl] = s
                elif j == 1:
                    den_v[sl] = jnp.exp(s)
                else:
                    den_v[sl] = den_v[sl] + jnp.exp(s)
                return 0
            lax.fori_loop(0, NG, score_grp, 0)

        pltpu.sync_copy(den_v, den_out.at[w])
        pltpu.sync_copy(sc_v, sc_out.at[w])

    return _main


def _finish_body(den_ref, sc_ref, out_ref):
    nll = (jnp.sum(jnp.log(den_ref[...])) - jnp.sum(sc_ref[...])) / B
    out_ref[0, 0] = nll


def kernel(inputs, predict, normal, I_H, H_U):
    ih_flat = I_H.T.reshape(V * D)
    hu_flat = H_U.T.reshape(V * D)
    ii = inputs.reshape(NW, BW).astype(jnp.int32)
    hui = jnp.concatenate(
        [predict.reshape(B, 1), normal.reshape(B, K)], axis=1
    ).astype(jnp.int32)
    hui = hui.reshape(NW, BW, S).transpose(0, 2, 1)
    den, scp = _make_main_sc()(ih_flat, hu_flat, ii, hui)
    out = pl.pallas_call(
        _finish_body,
        out_shape=jax.ShapeDtypeStruct((1, 1), jnp.float32),
        out_specs=pl.BlockSpec(memory_space=pltpu.SMEM),
    )(den, scp)
    return out.reshape(1)


# speedup vs baseline: 1.0035x; 1.0034x over previous
import functools

import jax
import jax.numpy as jnp
from jax import lax
from jax.experimental import pallas as pl
from jax.experimental.pallas import tpu as pltpu
from jax.experimental.pallas import tpu_sc as plsc

V = 1000000
B = 4096
K = 20
D = 16
S = K + 1

NC = 2
NS = 16
NW = NC * NS
BW = B // NW
NG = BW // 16
NR = BW * D // 128


def _make_main_sc():
    mesh = plsc.VectorSubcoreMesh(core_axis_name="c", subcore_axis_name="s")

    @functools.partial(
        pl.kernel,
        mesh=mesh,
        out_type=[
            jax.ShapeDtypeStruct((NW, BW), jnp.float32),
            jax.ShapeDtypeStruct((NW, BW), jnp.float32),
        ],
        scratch_types=[
            pltpu.VMEM((BW,), jnp.int32),
            pltpu.VMEM((S, BW), jnp.int32),
            pltpu.VMEM((BW * D,), jnp.int32),
            pltpu.VMEM((BW * D,), jnp.int32),
            pltpu.VMEM((BW * D,), jnp.int32),
            pltpu.VMEM((BW * D, D), jnp.float32),
            pltpu.VMEM((BW * D, D), jnp.float32),
            pltpu.VMEM((BW * D, D), jnp.float32),
            pltpu.VMEM((D, BW), jnp.float32),
            pltpu.VMEM((BW,), jnp.int32),
            pltpu.VMEM((S, BW), jnp.int32),
            pltpu.VMEM((BW,), jnp.float32),
            pltpu.VMEM((BW,), jnp.float32),
            pltpu.SemaphoreType.DMA,
            pltpu.SemaphoreType.DMA,
            pltpu.SemaphoreType.DMA,
        ],
        compiler_params=pltpu.CompilerParams(
            needs_layout_passes=False, use_tc_tiling_on_sc=False),
    )
    def _main(ih_flat, hu_flat, ii_hbm, hui_hbm, den_out, sc_out,
              ii_v, hui_v, idxe, idx0, idx1, gie, gb0, gb1,
              ieT_v, iln_v, hln_v, den_v, sc_v, sem_ie, sem0, sem1):
        w = lax.axis_index("s") * NC + lax.axis_index("c")
        pltpu.sync_copy(ii_hbm.at[w], ii_v)
        pltpu.sync_copy(hui_hbm.at[w], hui_v)

        VD = V // D

        def build_idx(idx_ref, vec_of):
            def bc(c, _):
                base = lax.shift_right_logical(vec_of(c), 4)
                for d in range(D):
                    idx_ref[pl.ds(c * 256 + d * 16, 16)] = base + d * VD
                return 0
            lax.fori_loop(0, NG, bc, 0)

        def build_lanes(ln_ref):
            def bl(t, _):
                sl = pl.ds(t * 16, 16)
                iln_v[sl] = ii_v[sl] & 15
                return 0
            lax.fori_loop(0, NG, bl, 0)
            def bh(t, _):
                j = t // NG
                c = t % NG
                sl = pl.ds(c * 16, 16)
                hln_v[j, sl] = hui_v[j, sl] & 15
                return 0
            lax.fori_loop(0, S * NG, bh, 0)

        build_lanes(None)

        def fire(table, idx_ref, buf, sem):
            return [pltpu.async_copy(table.at[idx_ref], buf, sem)]

        lanes16 = lax.iota(jnp.int32, 16)
        build_idx(idxe, lambda c: ii_v[pl.ds(c * 16, 16)])
        cp_ie = fire(ih_flat, idxe, gie, sem_ie)
        idxs = (idx0, idx1)
        bufs = (gb0, gb1)
        sems = (sem0, sem1)
        build_idx(idxs[0], lambda c: hui_v[0, pl.ds(c * 16, 16)])
        cps = {0: fire(hu_flat, idxs[0], bufs[0], sems[0])}
        for c in cp_ie:
            c.wait()

        def ex_ie(c, _):
            sl = pl.ds(c * 16, 16)
            ln = iln_v[sl]
            for d in range(D):
                rows = c * 256 + d * 16 + lanes16
                ieT_v[d, sl] = plsc.load_gather(gie, [rows, ln])
            return 0
        lax.fori_loop(0, NG, ex_ie, 0)

        for j in range(S):
            if j + 1 < S:
                nb = (j + 1) % 2
                build_idx(idxs[nb],
                          lambda c, _j=j: hui_v[_j + 1, pl.ds(c * 16, 16)])
                cps[j + 1] = fire(hu_flat, idxs[nb], bufs[nb], sems[nb])
            for c in cps[j]:
                c.wait()
            buf = bufs[j % 2]

            def score_grp(c, _):
                sl = pl.ds(c * 16, 16)
                ln = hln_v[j, sl]
                rows = c * 256 + lanes16
                s = plsc.load_gather(buf, [rows, ln]) * ieT_v[0, sl]
                for d in range(1, D):
                    rows = c * 256 + d * 16 + lanes16
                    s = s + plsc.load_gather(buf, [rows, ln]) * ieT_v[d, sl]
                if j == 0:
                    sc_v[sl] = s
                elif j == 1:
                    den_v[sl] = jnp.exp(s)
                else:
                    den_v[sl] = den_v[sl] + jnp.exp(s)
                return 0
            lax.fori_loop(0, NG, score_grp, 0)

        pltpu.sync_copy(den_v, den_out.at[w])
        pltpu.sync_copy(sc_v, sc_out.at[w])

    return _main


def _finish_body(den_ref, sc_ref, out_ref):
    nll = (jnp.sum(jnp.log(den_ref[...])) - jnp.sum(sc_ref[...])) / B
    out_ref[0, 0] = nll


def kernel(inputs, predict, normal, I_H, H_U):
    ih_flat = I_H.T.reshape(V * D).reshape(V, D)
    hu_flat = H_U.T.reshape(V * D).reshape(V, D)
    ii = inputs.reshape(NW, BW).astype(jnp.int32)
    hui = jnp.concatenate(
        [predict.reshape(B, 1), normal.reshape(B, K)], axis=1
    ).astype(jnp.int32)
    hui = hui.reshape(NW, BW, S).transpose(0, 2, 1)
    den, scp = _make_main_sc()(ih_flat, hu_flat, ii, hui)
    out = pl.pallas_call(
        _finish_body,
        out_shape=jax.ShapeDtypeStruct((1, 1), jnp.float32),
        out_specs=pl.BlockSpec(memory_space=pltpu.SMEM),
    )(den, scp)
    return out.reshape(1)
